# Initial kernel scaffold; baseline (speedup 1.0000x reference)
import jax
import jax.numpy as jnp
from jax.experimental import pallas as pl


def kernel(pred_logits, pred_boxes, target_sizes):
    raise NotImplementedError("write your pallas kernel here")


# speedup vs baseline: 2.7680x; 2.7680x over previous
import functools

import jax
import jax.numpy as jnp
from jax import lax
from jax.experimental import pallas as pl
from jax.experimental.pallas import tpu as pltpu
from jax.experimental.pallas import tpu_sc as plsc

B = 16
N = 20000
LANES = 16
NB = N // LANES
NBINS = 256
HSIZE = NBINS * LANES


def _prep_body(lt_ref, bt_ref, ts_ref, key_ref, meta_ref, w_ref):
    l0 = lt_ref[0]
    best = l0
    cls = jnp.zeros(l0.shape, jnp.int32)
    for c in (1, 2, 3):
        lc = lt_ref[c]
        gt = lc > best
        cls = jnp.where(gt, jnp.int32(c), cls)
        best = jnp.where(gt, lc, best)
    bg = cls == 0
    center = jnp.where(bg, jnp.float32(0.0), bt_ref[0])
    absw = jnp.where(bg, jnp.float32(0.0), jnp.abs(bt_ref[1]))
    s = jnp.sum(absw, axis=1, keepdims=True)
    wfrac = absw / (s + jnp.float32(1e-8))
    w_ref[...] = (wfrac * ts_ref[...]).astype(jnp.int32)
    bits = lax.bitcast_convert_type(center, jnp.int32)
    key_ref[...] = jnp.where(bits >= 0, bits | jnp.int32(-(2 ** 31)), ~bits)
    iota = lax.broadcasted_iota(jnp.int32, (B, N), 1)
    meta_ref[...] = (iota << 2) | cls


def _sort_body(key_hbm, meta_hbm, w_hbm, cls_out, w_out, ka, ma, kb, mb, wv, hist):
    c = lax.axis_index("c")
    s = lax.axis_index("s")

    @pl.when(s < 8)
    def _():
        row = c * 8 + s
        pltpu.sync_copy(key_hbm.at[row], ka)
        pltpu.sync_copy(meta_hbm.at[row], ma)
        pltpu.sync_copy(w_hbm.at[row], wv)
        lane = lax.iota(jnp.int32, LANES)
        gidx = lane * NB
        ones = jnp.ones((LANES,), jnp.int32)
        zeros = jnp.zeros((LANES,), jnp.int32)

        for pno, (src_k, src_m, dst_k, dst_m) in enumerate((
                (ka, ma, kb, mb), (kb, mb, ka, ma),
                (ka, ma, kb, mb), (kb, mb, ka, ma))):
            shift = pno * 8

            def zbody(j, _):
                hist[pl.ds(j * LANES, LANES)] = zeros
                return 0
            lax.fori_loop(0, HSIZE // LANES, zbody, 0)

            def hbody(i, _, src_k=src_k, shift=shift):
                k = plsc.load_gather(src_k, [gidx + i])
                d = lax.shift_right_logical(k, shift) & 255
                ci = (d << 4) | lane
                plsc.addupdate_scatter(hist, [ci], ones)
                return 0
            lax.fori_loop(0, NB, hbody, 0)

            def sbody(j, carry):
                v = hist[pl.ds(j * LANES, LANES)]
                incl = plsc.cumsum(v)
                hist[pl.ds(j * LANES, LANES)] = (incl - v) + carry
                return carry + jnp.sum(v, axis=0)
            lax.fori_loop(0, HSIZE // LANES, sbody, jnp.int32(0))

            def pbody(i, _, src_k=src_k, src_m=src_m, dst_k=dst_k,
                      dst_m=dst_m, shift=shift):
                idx = gidx + i
                k = plsc.load_gather(src_k, [idx])
                m = plsc.load_gather(src_m, [idx])
                d = lax.shift_right_logical(k, shift) & 255
                ci = (d << 4) | lane
                pos = plsc.load_gather(hist, [ci])
                plsc.store_scatter(hist, [ci], pos + 1)
                plsc.store_scatter(dst_k, [pos], k)
                plsc.store_scatter(dst_m, [pos], m)
                return 0
            lax.fori_loop(0, NB, pbody, 0)

        def fbody(i, _):
            m = ma[pl.ds(i * LANES, LANES)]
            idx2 = lax.shift_right_logical(m, 2)
            w = plsc.load_gather(wv, [idx2])
            kb[pl.ds(i * LANES, LANES)] = m & 3
            mb[pl.ds(i * LANES, LANES)] = w
            return 0
        lax.fori_loop(0, NB, fbody, 0)

        pltpu.sync_copy(kb, cls_out.at[row])
        pltpu.sync_copy(mb, w_out.at[row])


@jax.jit
def kernel(pred_logits, pred_boxes, target_sizes):
    lt = jnp.transpose(pred_logits, (2, 0, 1))
    bt = jnp.transpose(pred_boxes, (2, 0, 1))
    ts = target_sizes.astype(jnp.float32)[:, None]

    key, meta, w = pl.pallas_call(
        _prep_body,
        out_shape=(
            jax.ShapeDtypeStruct((B, N), jnp.int32),
            jax.ShapeDtypeStruct((B, N), jnp.int32),
            jax.ShapeDtypeStruct((B, N), jnp.int32),
        ),
    )(lt, bt, ts)

    sort = pl.kernel(
        _sort_body,
        out_type=(
            jax.ShapeDtypeStruct((B, N), jnp.int32),
            jax.ShapeDtypeStruct((B, N), jnp.int32),
        ),
        mesh=plsc.VectorSubcoreMesh(core_axis_name="c", subcore_axis_name="s"),
        compiler_params=pltpu.CompilerParams(needs_layout_passes=False),
        scratch_types=[
            pltpu.VMEM((N,), jnp.int32),
            pltpu.VMEM((N,), jnp.int32),
            pltpu.VMEM((N,), jnp.int32),
            pltpu.VMEM((N,), jnp.int32),
            pltpu.VMEM((N,), jnp.int32),
            pltpu.VMEM((HSIZE,), jnp.int32),
        ],
    )
    cls_s, w_s = sort(key, meta, w)
    return (cls_s, w_s)
